# Initial kernel scaffold; baseline (speedup 1.0000x reference)
import jax
import jax.numpy as jnp
from jax.experimental import pallas as pl


def kernel(x, edge_index, manual_features, W1_self, W1_neigh, b1, W2_self, W2_neigh, b2, W3, b3, W4, b4):
    raise NotImplementedError("write your pallas kernel here")


# speedup vs baseline: 4.8170x; 4.8170x over previous
import functools

import jax
import jax.numpy as jnp
from jax import lax
from jax.experimental import pallas as pl
from jax.experimental.pallas import tpu as pltpu
from jax.experimental.pallas import tpu_sc as plsc

N = 10000
D = 128
E = 320000
MF = 16
NP = 10240
RPT = NP // 16
C = 80
NSC = 2
EPS = E // (NSC * 16)
NCH = EPS // C

_mesh = plsc.VectorSubcoreMesh(core_axis_name="c", subcore_axis_name="s")


def _sc_body(y_hbm, src_hbm, dst_hbm, agg_hbm, deg_hbm,
             src_v, dst_v, rows_v, zrow_v, zdeg_v, ones_v,
             acc_sh, deg_sh, sem):
    c = lax.axis_index("c")
    s = lax.axis_index("s")

    zero16 = jnp.zeros((16,), jnp.float32)
    one16 = jnp.ones((16,), jnp.float32)

    def zrow_loop(i, carry):
        zrow_v[i // 8, pl.ds((i % 8) * 16, 16)] = zero16
        return carry

    lax.fori_loop(0, C * (D // 16), zrow_loop, 0)

    def zdeg_loop(i, carry):
        zdeg_v[pl.ds(i * 16, 16)] = zero16
        return carry

    lax.fori_loop(0, RPT // 16, zdeg_loop, 0)

    def ones_loop(i, carry):
        ones_v[pl.ds(i * 16, 16)] = one16
        return carry

    lax.fori_loop(0, C // 16, ones_loop, 0)

    row0 = s * RPT
    for k in range(RPT // C):
        pltpu.sync_copy(zrow_v, acc_sh.at[pl.ds(row0 + k * C, C)])
    pltpu.sync_copy(zdeg_v, deg_sh.at[pl.ds(row0, RPT)])
    plsc.subcore_barrier()

    base = c * (E // NSC) + s * EPS

    def chunk(i, carry):
        b = base + i * C
        pltpu.sync_copy(src_hbm.at[pl.ds(b, C)], src_v)
        pltpu.sync_copy(dst_hbm.at[pl.ds(b, C)], dst_v)
        pltpu.async_copy(y_hbm.at[src_v], rows_v, sem).wait()
        pltpu.sync_copy(rows_v, acc_sh.at[dst_v], add=True)
        pltpu.sync_copy(ones_v, deg_sh.at[dst_v], add=True)
        return carry

    lax.fori_loop(0, NCH, chunk, 0)
    plsc.subcore_barrier()

    out0 = c * NP + row0
    pltpu.sync_copy(acc_sh.at[pl.ds(row0, RPT)], agg_hbm.at[pl.ds(out0, RPT)])
    pltpu.sync_copy(deg_sh.at[pl.ds(row0, RPT)], deg_hbm.at[pl.ds(out0, RPT)])


_sc_agg = pl.kernel(
    _sc_body,
    out_type=[
        jax.ShapeDtypeStruct((NSC * NP, D), jnp.float32),
        jax.ShapeDtypeStruct((NSC * NP,), jnp.float32),
    ],
    mesh=_mesh,
    scratch_types=[
        pltpu.VMEM((C,), jnp.int32),
        pltpu.VMEM((C,), jnp.int32),
        pltpu.VMEM((C, D), jnp.float32),
        pltpu.VMEM((C, D), jnp.float32),
        pltpu.VMEM((RPT,), jnp.float32),
        pltpu.VMEM((C,), jnp.float32),
        pltpu.VMEM_SHARED((NP, D), jnp.float32),
        pltpu.VMEM_SHARED((NP,), jnp.float32),
        pltpu.SemaphoreType.DMA,
    ],
)


BM = 2000


def _tc_in_body(x_ref, wn_ref, ws_ref, b_ref, y_ref, z_ref):
    xb = x_ref[...]
    y_ref[...] = jnp.dot(xb, wn_ref[...], preferred_element_type=jnp.float32)
    z_ref[...] = jnp.dot(xb, ws_ref[...], preferred_element_type=jnp.float32) + b_ref[...]


_tc_in = pl.pallas_call(
    _tc_in_body,
    grid=(N // BM,),
    in_specs=[
        pl.BlockSpec((BM, D), lambda i: (i, 0)),
        pl.BlockSpec((D, D), lambda i: (0, 0)),
        pl.BlockSpec((D, D), lambda i: (0, 0)),
        pl.BlockSpec((1, D), lambda i: (0, 0)),
    ],
    out_specs=[
        pl.BlockSpec((BM, D), lambda i: (i, 0)),
        pl.BlockSpec((BM, D), lambda i: (i, 0)),
    ],
    out_shape=[
        jax.ShapeDtypeStruct((N, D), jnp.float32),
        jax.ShapeDtypeStruct((N, D), jnp.float32),
    ],
)


def _h_from_parts(z, aggA, aggB, degA, degB):
    deg = jnp.maximum(degA + degB, 1.0)
    h = z + (aggA + aggB) / deg
    return jnp.where(h >= 0, h, 0.01 * h)


def _tc_mid_body(z_ref, aA_ref, aB_ref, dA_ref, dB_ref, wn_ref, ws_ref, b_ref,
                 y_ref, z2_ref):
    h = _h_from_parts(z_ref[...], aA_ref[...], aB_ref[...], dA_ref[...], dB_ref[...])
    y_ref[...] = jnp.dot(h, wn_ref[...], preferred_element_type=jnp.float32)
    z2_ref[...] = jnp.dot(h, ws_ref[...], preferred_element_type=jnp.float32) + b_ref[...]


_tc_mid = pl.pallas_call(
    _tc_mid_body,
    grid=(N // BM,),
    in_specs=[
        pl.BlockSpec((BM, D), lambda i: (i, 0)),
        pl.BlockSpec((BM, D), lambda i: (i, 0)),
        pl.BlockSpec((BM, D), lambda i: (i, 0)),
        pl.BlockSpec((BM, 1), lambda i: (i, 0)),
        pl.BlockSpec((BM, 1), lambda i: (i, 0)),
        pl.BlockSpec((D, D), lambda i: (0, 0)),
        pl.BlockSpec((D, D), lambda i: (0, 0)),
        pl.BlockSpec((1, D), lambda i: (0, 0)),
    ],
    out_specs=[
        pl.BlockSpec((BM, D), lambda i: (i, 0)),
        pl.BlockSpec((BM, D), lambda i: (i, 0)),
    ],
    out_shape=[
        jax.ShapeDtypeStruct((N, D), jnp.float32),
        jax.ShapeDtypeStruct((N, D), jnp.float32),
    ],
)


def _tc_fin_body(z_ref, aA_ref, aB_ref, dA_ref, dB_ref, sum_ref):
    h = _h_from_parts(z_ref[...], aA_ref[...], aB_ref[...], dA_ref[...], dB_ref[...])
    part = jnp.sum(h, axis=0, keepdims=True)

    @pl.when(pl.program_id(0) == 0)
    def _init():
        sum_ref[...] = part

    @pl.when(pl.program_id(0) != 0)
    def _acc():
        sum_ref[...] += part


_tc_fin = pl.pallas_call(
    _tc_fin_body,
    grid=(N // BM,),
    in_specs=[
        pl.BlockSpec((BM, D), lambda i: (i, 0)),
        pl.BlockSpec((BM, D), lambda i: (i, 0)),
        pl.BlockSpec((BM, D), lambda i: (i, 0)),
        pl.BlockSpec((BM, 1), lambda i: (i, 0)),
        pl.BlockSpec((BM, 1), lambda i: (i, 0)),
    ],
    out_specs=pl.BlockSpec((1, D), lambda i: (0, 0)),
    out_shape=jax.ShapeDtypeStruct((1, D), jnp.float32),
)


def _tc_head_body(s_ref, mf_ref, w3a_ref, w3b_ref, b3_ref, w4_ref, b4_ref, o_ref):
    g = s_ref[...] * (1.0 / N)
    t = (jnp.dot(g, w3a_ref[...], preferred_element_type=jnp.float32)
         + jnp.dot(mf_ref[...], w3b_ref[...], preferred_element_type=jnp.float32)
         + b3_ref[...])
    a = jnp.maximum(t, 0.0)
    o_ref[...] = jnp.dot(a, w4_ref[...], preferred_element_type=jnp.float32) + b4_ref[...]


def _tc_head(ssum, mf, w3a, w3b, b3, w4, b4):
    return pl.pallas_call(
        _tc_head_body,
        out_shape=jax.ShapeDtypeStruct((1, 1), jnp.float32),
    )(ssum, mf, w3a, w3b, b3, w4, b4)


def kernel(x, edge_index, manual_features, W1_self, W1_neigh, b1,
           W2_self, W2_neigh, b2, W3, b3, W4, b4):
    src = edge_index[0]
    dst = edge_index[1]

    y1, z1 = _tc_in(x, W1_neigh, W1_self, b1.reshape(1, D))
    agg1, deg = _sc_agg(y1, src, dst)
    aggA1, aggB1 = agg1[:N], agg1[NP:NP + N]
    degA = deg[:N].reshape(N, 1)
    degB = deg[NP:NP + N].reshape(N, 1)

    y2, z2 = _tc_mid(z1, aggA1, aggB1, degA, degB,
                     W2_neigh, W2_self, b2.reshape(1, D))
    agg2, _ = _sc_agg(y2, src, dst)

    ssum = _tc_fin(z2, agg2[:N], agg2[NP:NP + N], degA, degB)
    res = _tc_head(ssum, manual_features.reshape(1, MF),
                   W3[:D], W3[D:], b3.reshape(1, -1), W4, b4.reshape(1, 1))
    return res.reshape((1,))


# speedup vs baseline: 11.2739x; 2.3404x over previous
import functools

import jax
import jax.numpy as jnp
from jax import lax
from jax.experimental import pallas as pl
from jax.experimental.pallas import tpu as pltpu
from jax.experimental.pallas import tpu_sc as plsc

N = 10000
D = 128
E = 320000
MF = 16
NP = 10240
RPT = NP // 16
C = 80
NSC = 2
EPS = E // (NSC * 16)
NCH = EPS // C
NBUF = 2
NCHP = 128

_mesh = plsc.VectorSubcoreMesh(core_axis_name="c", subcore_axis_name="s")


def _sc_body(with_deg, y_hbm, pk_hbm, *rest):
    if with_deg:
        (agg_hbm, deg_hbm, packb, r0, r1, sv0, sv1, dv0, dv1, zdeg_v, ones_v,
         acc_sh, deg_sh, gsem, ssem, dsem) = rest
    else:
        (agg_hbm, packb, r0, r1, sv0, sv1, dv0, dv1,
         acc_sh, gsem, ssem) = rest
    rows = (r0, r1)
    srcv = (sv0, sv1)
    dstv = (dv0, dv1)

    c = lax.axis_index("c")
    s = lax.axis_index("s")

    zero16 = jnp.zeros((16,), jnp.float32)
    one16 = jnp.ones((16,), jnp.float32)

    for i in range(C):
        for k in range(D // 16):
            r0[i, pl.ds(k * 16, 16)] = zero16
    if with_deg:
        for i in range(C // 16):
            zdeg_v[pl.ds(i * 16, 16)] = zero16
            ones_v[pl.ds(i * 16, 16)] = one16

    crow0 = (c * 16 + s) * NCHP
    pltpu.sync_copy(pk_hbm.at[pl.ds(crow0, NCHP)], packb)

    row0 = s * RPT
    for k in range(RPT // C):
        pltpu.sync_copy(r0, acc_sh.at[pl.ds(row0 + k * C, C)])
        if with_deg:
            pltpu.sync_copy(zdeg_v, deg_sh.at[pl.ds(row0 + k * C, C)])
    plsc.subcore_barrier()

    def decode(g, b):
        for k in range(C // 16):
            v16 = packb[g, pl.ds(k * 16, 16)]
            srcv[b][pl.ds(k * 16, 16)] = lax.shift_right_logical(v16, 14)
            dstv[b][pl.ds(k * 16, 16)] = lax.bitwise_and(v16, 16383)

    def gather_start(g, b):
        decode(g, b)
        pltpu.async_copy(y_hbm.at[srcv[b]], rows[b], gsem.at[b])

    def gather_wait(b):
        pltpu.make_async_copy(y_hbm.at[srcv[b]], rows[b], gsem.at[b]).wait()

    def scatter_start(b):
        pltpu.async_copy(rows[b], acc_sh.at[dstv[b]], ssem.at[b], add=True)
        if with_deg:
            pltpu.async_copy(ones_v, deg_sh.at[dstv[b]], dsem.at[b], add=True)

    def scatter_wait(b):
        pltpu.make_async_copy(rows[b], acc_sh.at[dstv[b]], ssem.at[b]).wait()
        if with_deg:
            pltpu.make_async_copy(ones_v, deg_sh.at[dstv[b]], dsem.at[b]).wait()

    for b in range(NBUF):
        gather_start(b, b)

    def group(grp, carry):
        for b in range(NBUF):
            g = grp * NBUF + b
            gather_wait(b)
            scatter_start(b)
            scatter_wait(b)
            gn = jnp.minimum(g + NBUF, NCH - 1)
            gather_start(gn, b)
        return carry

    lax.fori_loop(0, (NCH - 1) // NBUF, group, 0)
    gather_wait(0)
    scatter_start(0)
    scatter_wait(0)
    gather_wait(1)
    plsc.subcore_barrier()

    pltpu.sync_copy(acc_sh.at[pl.ds(row0, RPT)], agg_hbm.at[c, pl.ds(row0, RPT)])
    if with_deg:
        pltpu.sync_copy(deg_sh.at[pl.ds(row0, RPT)], deg_hbm.at[c, pl.ds(row0, RPT)])


_sc_agg_deg = pl.kernel(
    functools.partial(_sc_body, True),
    out_type=[
        jax.ShapeDtypeStruct((NSC, NP, D), jnp.float32),
        jax.ShapeDtypeStruct((NSC, NP), jnp.float32),
    ],
    mesh=_mesh,
    scratch_types=[
        pltpu.VMEM((NCHP, C), jnp.int32),
        pltpu.VMEM((C, D), jnp.float32),
        pltpu.VMEM((C, D), jnp.float32),
        pltpu.VMEM((C,), jnp.int32),
        pltpu.VMEM((C,), jnp.int32),
        pltpu.VMEM((C,), jnp.int32),
        pltpu.VMEM((C,), jnp.int32),
        pltpu.VMEM((C,), jnp.float32),
        pltpu.VMEM((C,), jnp.float32),
        pltpu.VMEM_SHARED((NP, D), jnp.float32),
        pltpu.VMEM_SHARED((NP,), jnp.float32),
        pltpu.SemaphoreType.DMA((NBUF,)),
        pltpu.SemaphoreType.DMA((NBUF,)),
        pltpu.SemaphoreType.DMA((NBUF,)),
    ],
)

_sc_agg = pl.kernel(
    functools.partial(_sc_body, False),
    out_type=jax.ShapeDtypeStruct((NSC, NP, D), jnp.float32),
    mesh=_mesh,
    scratch_types=[
        pltpu.VMEM((NCHP, C), jnp.int32),
        pltpu.VMEM((C, D), jnp.float32),
        pltpu.VMEM((C, D), jnp.float32),
        pltpu.VMEM((C,), jnp.int32),
        pltpu.VMEM((C,), jnp.int32),
        pltpu.VMEM((C,), jnp.int32),
        pltpu.VMEM((C,), jnp.int32),
        pltpu.VMEM_SHARED((NP, D), jnp.float32),
        pltpu.SemaphoreType.DMA((NBUF,)),
        pltpu.SemaphoreType.DMA((NBUF,)),
    ],
)


BM = 2000


def _tc_in_body(x_ref, wn_ref, ws_ref, b_ref, y_ref, z_ref):
    xb = x_ref[...]
    y_ref[...] = jnp.dot(xb, wn_ref[...], preferred_element_type=jnp.float32)
    z_ref[...] = jnp.dot(xb, ws_ref[...], preferred_element_type=jnp.float32) + b_ref[...]


_tc_in = pl.pallas_call(
    _tc_in_body,
    grid=(N // BM,),
    in_specs=[
        pl.BlockSpec((BM, D), lambda i: (i, 0)),
        pl.BlockSpec((D, D), lambda i: (0, 0)),
        pl.BlockSpec((D, D), lambda i: (0, 0)),
        pl.BlockSpec((1, D), lambda i: (0, 0)),
    ],
    out_specs=[
        pl.BlockSpec((BM, D), lambda i: (i, 0)),
        pl.BlockSpec((BM, D), lambda i: (i, 0)),
    ],
    out_shape=[
        jax.ShapeDtypeStruct((N, D), jnp.float32),
        jax.ShapeDtypeStruct((N, D), jnp.float32),
    ],
)


def _h_from_parts(z, aA, aB, dA, dB):
    deg = jnp.maximum(dA + dB, 1.0)
    h = z + (aA + aB) / deg
    return jnp.where(h >= 0, h, 0.01 * h)


_AGG_SPECS = [
    pl.BlockSpec((1, BM, D), lambda i: (0, i, 0)),
    pl.BlockSpec((1, BM, D), lambda i: (1, i, 0)),
    pl.BlockSpec((1, BM, 1), lambda i: (0, i, 0)),
    pl.BlockSpec((1, BM, 1), lambda i: (1, i, 0)),
]


def _tc_mid_body(z_ref, aA_ref, aB_ref, dA_ref, dB_ref, wn_ref, ws_ref, b_ref,
                 y_ref, z2_ref):
    h = _h_from_parts(z_ref[...], aA_ref[0], aB_ref[0], dA_ref[0], dB_ref[0])
    y_ref[...] = jnp.dot(h, wn_ref[...], preferred_element_type=jnp.float32)
    z2_ref[...] = jnp.dot(h, ws_ref[...], preferred_element_type=jnp.float32) + b_ref[...]


_tc_mid = pl.pallas_call(
    _tc_mid_body,
    grid=(N // BM,),
    in_specs=[pl.BlockSpec((BM, D), lambda i: (i, 0))] + _AGG_SPECS + [
        pl.BlockSpec((D, D), lambda i: (0, 0)),
        pl.BlockSpec((D, D), lambda i: (0, 0)),
        pl.BlockSpec((1, D), lambda i: (0, 0)),
    ],
    out_specs=[
        pl.BlockSpec((BM, D), lambda i: (i, 0)),
        pl.BlockSpec((BM, D), lambda i: (i, 0)),
    ],
    out_shape=[
        jax.ShapeDtypeStruct((N, D), jnp.float32),
        jax.ShapeDtypeStruct((N, D), jnp.float32),
    ],
)


def _tc_fin_body(z_ref, aA_ref, aB_ref, dA_ref, dB_ref, sum_ref):
    h = _h_from_parts(z_ref[...], aA_ref[0], aB_ref[0], dA_ref[0], dB_ref[0])
    part = jnp.sum(h, axis=0, keepdims=True)

    @pl.when(pl.program_id(0) == 0)
    def _init():
        sum_ref[...] = part

    @pl.when(pl.program_id(0) != 0)
    def _acc():
        sum_ref[...] += part


_tc_fin = pl.pallas_call(
    _tc_fin_body,
    grid=(N // BM,),
    in_specs=[pl.BlockSpec((BM, D), lambda i: (i, 0))] + _AGG_SPECS,
    out_specs=pl.BlockSpec((1, D), lambda i: (0, 0)),
    out_shape=jax.ShapeDtypeStruct((1, D), jnp.float32),
)


def _tc_head_body(s_ref, mf_ref, w3a_ref, w3b_ref, b3_ref, w4_ref, b4_ref, o_ref):
    g = s_ref[...] * (1.0 / N)
    t = (jnp.dot(g, w3a_ref[...], preferred_element_type=jnp.float32)
         + jnp.dot(mf_ref[...], w3b_ref[...], preferred_element_type=jnp.float32)
         + b3_ref[...])
    a = jnp.maximum(t, 0.0)
    o_ref[...] = jnp.dot(a, w4_ref[...], preferred_element_type=jnp.float32) + b4_ref[...]


def _tc_head(ssum, mf, w3a, w3b, b3, w4, b4):
    return pl.pallas_call(
        _tc_head_body,
        out_shape=jax.ShapeDtypeStruct((1, 1), jnp.float32),
    )(ssum, mf, w3a, w3b, b3, w4, b4)


def kernel(x, edge_index, manual_features, W1_self, W1_neigh, b1,
           W2_self, W2_neigh, b2, W3, b3, W4, b4):
    packed = edge_index[0] * 16384 + edge_index[1]
    pk2 = jnp.pad(packed.reshape(32, NCH, C),
                  ((0, 0), (0, NCHP - NCH), (0, 0))).reshape(32 * NCHP, C)

    y1, z1 = _tc_in(x, W1_neigh, W1_self, b1.reshape(1, D))
    agg1, deg = _sc_agg_deg(y1, pk2)
    deg3 = deg.reshape(NSC, NP, 1)

    y2, z2 = _tc_mid(z1, agg1, agg1, deg3, deg3,
                     W2_neigh, W2_self, b2.reshape(1, D))
    agg2 = _sc_agg(y2, pk2)

    ssum = _tc_fin(z2, agg2, agg2, deg3, deg3)
    res = _tc_head(ssum, manual_features.reshape(1, MF),
                   W3[:D], W3[D:], b3.reshape(1, -1), W4, b4.reshape(1, 1))
    return res.reshape((1,))
